# baseline (speedup 1.0000x reference)
import functools

import jax
import jax.numpy as jnp
from jax import lax
from jax.experimental import pallas as pl
from jax.experimental.pallas import tpu as pltpu
from jax.experimental.pallas import tpu_sc as plsc

_N_IMAGES = 150000
_BATCH = 16384
_ROT_D = 36
_TRA_D = 18
_ROT_S = 64
_TRA_S = 32
_L = 16
_CHUNK = 128

_NW = 32
_B_PER_W = _BATCH // _NW
_NCHUNK = _B_PER_W // _CHUNK


def _make_gather():
    mesh = plsc.VectorSubcoreMesh(core_axis_name="c", subcore_axis_name="s")

    @functools.partial(
        pl.kernel,
        mesh=mesh,
        compiler_params=pltpu.CompilerParams(
            use_tc_tiling_on_sc=False, needs_layout_passes=False),
        out_type=[
            jax.ShapeDtypeStruct((6, 8, _BATCH), jnp.float32),
            jax.ShapeDtypeStruct((6, 4, _BATCH), jnp.float32),
        ],
        scratch_types=[
            pltpu.VMEM((_NCHUNK, _CHUNK), jnp.int32),
            pltpu.VMEM((_B_PER_W, 8, 8), jnp.float32),
            pltpu.VMEM((_B_PER_W, 8, 8), jnp.float32),
            pltpu.VMEM((_ROT_D * _B_PER_W,), jnp.float32),
            pltpu.VMEM((_TRA_D * _B_PER_W,), jnp.float32),
            pltpu.SemaphoreType.DMA,
        ],
    )
    def gather_kernel(rot_hbm, tra_hbm, idx_hbm, rot_out, tra_out,
                      idx_v, rot_stage, tra_stage, rot_pack, tra_pack, sem):
        wid = lax.axis_index("s") * 2 + lax.axis_index("c")
        base = wid * _B_PER_W
        iota = lax.iota(jnp.int32, _L)

        pltpu.sync_copy(idx_hbm.at[pl.ds(wid * _NCHUNK, _NCHUNK)], idx_v)
        copies = []
        for c in range(_NCHUNK):
            copies.append(pltpu.async_copy(
                rot_hbm.at[idx_v.at[c]],
                rot_stage.at[pl.ds(c * _CHUNK, _CHUNK)], sem))
            copies.append(pltpu.async_copy(
                tra_hbm.at[idx_v.at[c]],
                tra_stage.at[pl.ds(c * _CHUNK, _CHUNK)], sem))
        for cp in copies:
            cp.wait()

        def rot_body(it, _):
            g = it & 31
            s = it >> 5
            d = (s * 10923) >> 16
            c = s - ((d << 2) + (d << 1))
            j = iota + g * _L
            vals = plsc.load_gather(
                rot_stage, [j, iota * 0 + d, iota * 0 + c])
            rot_pack[pl.ds(s * _B_PER_W + g * _L, _L)] = vals
            return 0

        lax.fori_loop(0, _ROT_D * (_B_PER_W // _L), rot_body, 0)

        def tra_body(it, _):
            g = it & 31
            s = it >> 5
            d = (s * 10923) >> 15
            c = s - ((d << 1) + d)
            j = iota + g * _L
            vals = plsc.load_gather(
                tra_stage, [j, iota * 0 + d, iota * 0 + c])
            tra_pack[pl.ds(s * _B_PER_W + g * _L, _L)] = vals
            return 0

        lax.fori_loop(0, _TRA_D * (_B_PER_W // _L), tra_body, 0)

        for s in range(_ROT_D):
            d, c = divmod(s, 6)
            pltpu.sync_copy(rot_pack.at[pl.ds(s * _B_PER_W, _B_PER_W)],
                            rot_out.at[d, c, pl.ds(base, _B_PER_W)])
        for s in range(_TRA_D):
            d, c = divmod(s, 3)
            pltpu.sync_copy(tra_pack.at[pl.ds(s * _B_PER_W, _B_PER_W)],
                            tra_out.at[d, c, pl.ds(base, _B_PER_W)])

    return gather_kernel


_GATHER = _make_gather()


def kernel(rotation_per_domain, translation_per_domain, indexes):
    n, d, _ = rotation_per_domain.shape
    rot_rows = jnp.pad(rotation_per_domain, ((0, 0), (0, 2), (0, 2)))
    tra_rows = jnp.pad(translation_per_domain, ((0, 0), (0, 2), (0, 5)))
    idx = indexes.astype(jnp.int32).reshape(_NW * _NCHUNK, _CHUNK)
    rot_o, tra_o = _GATHER(rot_rows, tra_rows, idx)
    rot = rot_o[:, :6, :].transpose(2, 0, 1)
    tra = tra_o[:, :3, :].transpose(2, 0, 1)
    return (rot, tra)


# speedup vs baseline: 18.7669x; 18.7669x over previous
import functools

import jax
import jax.numpy as jnp
from jax import lax
from jax.experimental import pallas as pl
from jax.experimental.pallas import tpu as pltpu
from jax.experimental.pallas import tpu_sc as plsc

_N_IMAGES = 150000
_BATCH = 16384
_ROT_D = 36
_TRA_D = 18
_ROT_S = 128
_TRA_S = 128
_L = 16
_CHUNK = 128

_NW = 32
_B_PER_W = _BATCH // _NW
_NCHUNK = _B_PER_W // _CHUNK


def _make_gather():
    mesh = plsc.VectorSubcoreMesh(core_axis_name="c", subcore_axis_name="s")

    @functools.partial(
        pl.kernel,
        mesh=mesh,
        compiler_params=pltpu.CompilerParams(
            use_tc_tiling_on_sc=False, needs_layout_passes=False),
        out_type=[
            jax.ShapeDtypeStruct((6, 8, _BATCH), jnp.float32),
            jax.ShapeDtypeStruct((6, 4, _BATCH), jnp.float32),
        ],
        scratch_types=[
            pltpu.VMEM((_NCHUNK, _CHUNK), jnp.int32),
            pltpu.VMEM((_B_PER_W // 2, _ROT_S), jnp.float32),
            pltpu.VMEM((_B_PER_W // 2, _TRA_S), jnp.float32),
            pltpu.VMEM((_ROT_D * _B_PER_W,), jnp.float32),
            pltpu.VMEM((_TRA_D * _B_PER_W,), jnp.float32),
            pltpu.SemaphoreType.DMA,
        ],
    )
    def gather_kernel(rot_hbm, tra_hbm, idx_hbm, rot_out, tra_out,
                      idx_v, rot_stage, tra_stage, rot_pack, tra_pack, sem):
        wid = lax.axis_index("s") * 2 + lax.axis_index("c")
        base = wid * _B_PER_W
        iota = lax.iota(jnp.int32, _L)

        pltpu.sync_copy(idx_hbm.at[pl.ds(wid * _NCHUNK, _NCHUNK)], idx_v)

        for hb in range(2):
            copies = []
            for c in range(_NCHUNK // 2):
                cc = hb * 2 + c
                copies.append(pltpu.async_copy(
                    rot_hbm.at[idx_v.at[cc]],
                    rot_stage.at[pl.ds(c * _CHUNK, _CHUNK)], sem))
                copies.append(pltpu.async_copy(
                    tra_hbm.at[idx_v.at[cc]],
                    tra_stage.at[pl.ds(c * _CHUNK, _CHUNK)], sem))
            for cp in copies:
                cp.wait()

            def rot_body(it, _):
                g = it & 15
                s = it >> 4
                j = iota + g * _L
                vals = plsc.load_gather(rot_stage, [j, iota * 0 + s])
                rot_pack[pl.ds(s * _B_PER_W + hb * 256 + g * _L, _L)] = vals
                return 0

            lax.fori_loop(0, _ROT_D * 16, rot_body, 0)

            def tra_body(it, _):
                g = it & 15
                s = it >> 4
                j = iota + g * _L
                vals = plsc.load_gather(tra_stage, [j, iota * 0 + s])
                tra_pack[pl.ds(s * _B_PER_W + hb * 256 + g * _L, _L)] = vals
                return 0

            lax.fori_loop(0, _TRA_D * 16, tra_body, 0)

        for s in range(_ROT_D):
            d, c = divmod(s, 6)
            pltpu.sync_copy(rot_pack.at[pl.ds(s * _B_PER_W, _B_PER_W)],
                            rot_out.at[d, c, pl.ds(base, _B_PER_W)])
        for s in range(_TRA_D):
            d, c = divmod(s, 3)
            pltpu.sync_copy(tra_pack.at[pl.ds(s * _B_PER_W, _B_PER_W)],
                            tra_out.at[d, c, pl.ds(base, _B_PER_W)])

    return gather_kernel


_GATHER = _make_gather()


def kernel(rotation_per_domain, translation_per_domain, indexes):
    n, d, _ = rotation_per_domain.shape
    rot_rows = jnp.pad(rotation_per_domain.reshape(n, _ROT_D),
                       ((0, 0), (0, _ROT_S - _ROT_D)))
    tra_rows = jnp.pad(translation_per_domain.reshape(n, _TRA_D),
                       ((0, 0), (0, _TRA_S - _TRA_D)))
    idx = indexes.astype(jnp.int32).reshape(_NW * _NCHUNK, _CHUNK)
    rot_o, tra_o = _GATHER(rot_rows, tra_rows, idx)
    rot = rot_o[:, :6, :].transpose(2, 0, 1)
    tra = tra_o[:, :3, :].transpose(2, 0, 1)
    return (rot, tra)


# speedup vs baseline: 19.2427x; 1.0254x over previous
import functools

import jax
import jax.numpy as jnp
from jax import lax
from jax.experimental import pallas as pl
from jax.experimental.pallas import tpu as pltpu
from jax.experimental.pallas import tpu_sc as plsc

_N_IMAGES = 150000
_BATCH = 16384
_ROT_D = 36
_TRA_D = 18
_ROT_S = 128
_TRA_S = 128
_L = 16
_CHUNK = 128

_NW = 32
_B_PER_W = _BATCH // _NW
_NCHUNK = _B_PER_W // _CHUNK


def _make_gather():
    mesh = plsc.VectorSubcoreMesh(core_axis_name="c", subcore_axis_name="s")

    @functools.partial(
        pl.kernel,
        mesh=mesh,
        compiler_params=pltpu.CompilerParams(
            use_tc_tiling_on_sc=False, needs_layout_passes=False),
        out_type=[
            jax.ShapeDtypeStruct((6, 8, _BATCH), jnp.float32),
            jax.ShapeDtypeStruct((6, 4, _BATCH), jnp.float32),
        ],
        scratch_types=[
            pltpu.VMEM((_NCHUNK, _CHUNK), jnp.int32),
            pltpu.VMEM((_B_PER_W // 2, 128), jnp.float32),
            pltpu.VMEM((_ROT_D * _B_PER_W,), jnp.float32),
            pltpu.VMEM((_TRA_D * _B_PER_W,), jnp.float32),
            pltpu.SemaphoreType.DMA,
        ],
    )
    def gather_kernel(tab_hbm, idx_hbm, rot_out, tra_out,
                      idx_v, stage, rot_pack, tra_pack, sem):
        wid = lax.axis_index("s") * 2 + lax.axis_index("c")
        base = wid * _B_PER_W
        iota = lax.iota(jnp.int32, _L)

        pltpu.sync_copy(idx_hbm.at[pl.ds(wid * _NCHUNK, _NCHUNK)], idx_v)

        for hb in range(2):
            copies = []
            for c in range(_NCHUNK // 2):
                cc = hb * 2 + c
                copies.append(pltpu.async_copy(
                    tab_hbm.at[idx_v.at[cc]],
                    stage.at[pl.ds(c * _CHUNK, _CHUNK)], sem))
            for cp in copies:
                cp.wait()

            def rot_body(it, _):
                g = it & 15
                s = it >> 4
                j = iota + g * _L
                vals = plsc.load_gather(stage, [j, iota * 0 + s])
                rot_pack[pl.ds(s * _B_PER_W + hb * 256 + g * _L, _L)] = vals
                return 0

            lax.fori_loop(0, _ROT_D * 16, rot_body, 0)

            def tra_body(it, _):
                g = it & 15
                s = it >> 4
                j = iota + g * _L
                vals = plsc.load_gather(stage, [j, iota * 0 + s + _ROT_D])
                tra_pack[pl.ds(s * _B_PER_W + hb * 256 + g * _L, _L)] = vals
                return 0

            lax.fori_loop(0, _TRA_D * 16, tra_body, 0)

        for s in range(_ROT_D):
            d, c = divmod(s, 6)
            pltpu.sync_copy(rot_pack.at[pl.ds(s * _B_PER_W, _B_PER_W)],
                            rot_out.at[d, c, pl.ds(base, _B_PER_W)])
        for s in range(_TRA_D):
            d, c = divmod(s, 3)
            pltpu.sync_copy(tra_pack.at[pl.ds(s * _B_PER_W, _B_PER_W)],
                            tra_out.at[d, c, pl.ds(base, _B_PER_W)])

    return gather_kernel


_GATHER = _make_gather()


def kernel(rotation_per_domain, translation_per_domain, indexes):
    n, d, _ = rotation_per_domain.shape
    tab = jnp.pad(
        jnp.concatenate([rotation_per_domain.reshape(n, _ROT_D),
                         translation_per_domain.reshape(n, _TRA_D)], axis=1),
        ((0, 0), (0, 128 - _ROT_D - _TRA_D)))
    idx = indexes.astype(jnp.int32).reshape(_NW * _NCHUNK, _CHUNK)
    rot_o, tra_o = _GATHER(tab, idx)
    rot = rot_o[:, :6, :].transpose(2, 0, 1)
    tra = tra_o[:, :3, :].transpose(2, 0, 1)
    return (rot, tra)


# speedup vs baseline: 19.6616x; 1.0218x over previous
import functools

import jax
import jax.numpy as jnp
from jax import lax
from jax.experimental import pallas as pl
from jax.experimental.pallas import tpu as pltpu
from jax.experimental.pallas import tpu_sc as plsc

_N_IMAGES = 150000
_BATCH = 16384
_ROT_D = 36
_TRA_D = 18
_ROT_S = 128
_TRA_S = 128
_L = 16
_CHUNK = 128

_NW = 32
_B_PER_W = _BATCH // _NW
_NCHUNK = _B_PER_W // _CHUNK


def _make_gather():
    mesh = plsc.VectorSubcoreMesh(core_axis_name="c", subcore_axis_name="s")

    @functools.partial(
        pl.kernel,
        mesh=mesh,
        compiler_params=pltpu.CompilerParams(
            use_tc_tiling_on_sc=False, needs_layout_passes=False),
        out_type=[
            jax.ShapeDtypeStruct((6, 8, _BATCH), jnp.float32),
            jax.ShapeDtypeStruct((6, 4, _BATCH), jnp.float32),
        ],
        scratch_types=[
            pltpu.VMEM((_NCHUNK, _CHUNK), jnp.int32),
            pltpu.VMEM((_B_PER_W // 2, 128), jnp.float32),
            pltpu.VMEM((_ROT_D * _B_PER_W,), jnp.float32),
            pltpu.VMEM((_TRA_D * _B_PER_W,), jnp.float32),
            pltpu.SemaphoreType.DMA,
        ],
    )
    def gather_kernel(tab_hbm, idx_hbm, rot_out, tra_out,
                      idx_v, stage, rot_pack, tra_pack, sem):
        wid = lax.axis_index("s") * 2 + lax.axis_index("c")
        base = wid * _B_PER_W
        iota = lax.iota(jnp.int32, _L)

        pltpu.sync_copy(idx_hbm.at[pl.ds(wid * _NCHUNK, _NCHUNK)], idx_v)

        for hb in range(2):
            copies = []
            for c in range(_NCHUNK // 2):
                cc = hb * 2 + c
                copies.append(pltpu.async_copy(
                    tab_hbm.at[idx_v.at[cc]],
                    stage.at[pl.ds(c * _CHUNK, _CHUNK)], sem))
            for cp in copies:
                cp.wait()

            def rot_body(it, _):
                g = it & 15
                s = it >> 4
                j = iota + g * _L
                vals = plsc.load_gather(stage, [j, iota * 0 + s])
                rot_pack[pl.ds(s * _B_PER_W + hb * 256 + g * _L, _L)] = vals
                return 0

            lax.fori_loop(0, _ROT_D * 16, rot_body, 0)

            def tra_body(it, _):
                g = it & 15
                s = it >> 4
                j = iota + g * _L
                vals = plsc.load_gather(stage, [j, iota * 0 + s + _ROT_D])
                tra_pack[pl.ds(s * _B_PER_W + hb * 256 + g * _L, _L)] = vals
                return 0

            lax.fori_loop(0, _TRA_D * 16, tra_body, 0)

        for s in range(_ROT_D):
            d, c = divmod(s, 6)
            pltpu.sync_copy(rot_pack.at[pl.ds(s * _B_PER_W, _B_PER_W)],
                            rot_out.at[d, c, pl.ds(base, _B_PER_W)])
        for s in range(_TRA_D):
            d, c = divmod(s, 3)
            pltpu.sync_copy(tra_pack.at[pl.ds(s * _B_PER_W, _B_PER_W)],
                            tra_out.at[d, c, pl.ds(base, _B_PER_W)])

    return gather_kernel


_GATHER = _make_gather()


def kernel(rotation_per_domain, translation_per_domain, indexes):
    n, d, _ = rotation_per_domain.shape
    tab = jnp.concatenate(
        [rotation_per_domain.reshape(n, _ROT_D),
         translation_per_domain.reshape(n, _TRA_D),
         jnp.zeros((n, 128 - _ROT_D - _TRA_D), jnp.float32)], axis=1)
    idx = indexes.astype(jnp.int32).reshape(_NW * _NCHUNK, _CHUNK)
    rot_o, tra_o = _GATHER(tab, idx)
    rot = rot_o[:, :6, :].transpose(2, 0, 1)
    tra = tra_o[:, :3, :].transpose(2, 0, 1)
    return (rot, tra)


# speedup vs baseline: 19.8520x; 1.0097x over previous
import functools

import jax
import jax.numpy as jnp
from jax import lax
from jax.experimental import pallas as pl
from jax.experimental.pallas import tpu as pltpu
from jax.experimental.pallas import tpu_sc as plsc

_N_IMAGES = 150000
_BATCH = 16384
_ROT_D = 36
_TRA_D = 18
_ROT_S = 128
_TRA_S = 128
_L = 16
_CHUNK = 128

_NW = 32
_B_PER_W = _BATCH // _NW
_NCHUNK = _B_PER_W // _CHUNK


def _make_gather():
    mesh = plsc.VectorSubcoreMesh(core_axis_name="c", subcore_axis_name="s")

    @functools.partial(
        pl.kernel,
        mesh=mesh,
        compiler_params=pltpu.CompilerParams(
            use_tc_tiling_on_sc=False, needs_layout_passes=False),
        out_type=[
            jax.ShapeDtypeStruct((6, 8, _BATCH), jnp.float32),
            jax.ShapeDtypeStruct((6, 4, _BATCH), jnp.float32),
        ],
        scratch_types=[
            pltpu.VMEM((_NCHUNK, _CHUNK), jnp.int32),
            pltpu.VMEM((2, _B_PER_W // 2, 128), jnp.float32),
            pltpu.VMEM((_ROT_D * _B_PER_W,), jnp.float32),
            pltpu.VMEM((_TRA_D * _B_PER_W,), jnp.float32),
            pltpu.SemaphoreType.DMA,
        ],
    )
    def gather_kernel(tab_hbm, idx_hbm, rot_out, tra_out,
                      idx_v, stage, rot_pack, tra_pack, sem):
        wid = lax.axis_index("s") * 2 + lax.axis_index("c")
        base = wid * _B_PER_W
        iota = lax.iota(jnp.int32, _L)

        pltpu.sync_copy(idx_hbm.at[pl.ds(wid * _NCHUNK, _NCHUNK)], idx_v)

        def fire(hb):
            return [pltpu.async_copy(
                        tab_hbm.at[idx_v.at[hb * 2 + c]],
                        stage.at[hb, pl.ds(c * _CHUNK, _CHUNK)], sem)
                    for c in range(_NCHUNK // 2)]

        pending = fire(0)
        for hb in range(2):
            for cp in pending:
                cp.wait()
            if hb == 0:
                pending = fire(1)

            def rot_body(it, _):
                g = it & 15
                s = it >> 4
                j = iota + g * _L
                vals = plsc.load_gather(stage, [iota * 0 + hb, j, iota * 0 + s])
                rot_pack[pl.ds(s * _B_PER_W + hb * 256 + g * _L, _L)] = vals
                return 0

            lax.fori_loop(0, _ROT_D * 16, rot_body, 0)

            def tra_body(it, _):
                g = it & 15
                s = it >> 4
                j = iota + g * _L
                vals = plsc.load_gather(stage, [iota * 0 + hb, j, iota * 0 + s + _ROT_D])
                tra_pack[pl.ds(s * _B_PER_W + hb * 256 + g * _L, _L)] = vals
                return 0

            lax.fori_loop(0, _TRA_D * 16, tra_body, 0)

        for s in range(_ROT_D):
            d, c = divmod(s, 6)
            pltpu.sync_copy(rot_pack.at[pl.ds(s * _B_PER_W, _B_PER_W)],
                            rot_out.at[d, c, pl.ds(base, _B_PER_W)])
        for s in range(_TRA_D):
            d, c = divmod(s, 3)
            pltpu.sync_copy(tra_pack.at[pl.ds(s * _B_PER_W, _B_PER_W)],
                            tra_out.at[d, c, pl.ds(base, _B_PER_W)])

    return gather_kernel


_GATHER = _make_gather()


def kernel(rotation_per_domain, translation_per_domain, indexes):
    n, d, _ = rotation_per_domain.shape
    tab = jnp.concatenate(
        [rotation_per_domain.reshape(n, _ROT_D),
         translation_per_domain.reshape(n, _TRA_D),
         jnp.zeros((n, 128 - _ROT_D - _TRA_D), jnp.float32)], axis=1)
    idx = indexes.astype(jnp.int32).reshape(_NW * _NCHUNK, _CHUNK)
    rot_o, tra_o = _GATHER(tab, idx)
    rot = rot_o[:, :6, :].transpose(2, 0, 1)
    tra = tra_o[:, :3, :].transpose(2, 0, 1)
    return (rot, tra)
